# baseline (speedup 1.0000x reference)
import functools

import numpy as np
import jax
import jax.numpy as jnp
from jax.experimental import pallas as pl
from jax.experimental.pallas import tpu as pltpu
from jax.sharding import Mesh, PartitionSpec as P

_CH = 8
_PAD = 1


def _geometry(t_shapes):
    plan = []
    cin = 1
    for (k, wpcin, wocout) in t_shapes:
        wp = wpcin // cin
        wo = wocout // _CH
        win = wp - 2 * _PAD
        ho = wp - k + 1
        assert ho == wo
        plan.append(dict(k=k, cin=cin, win=win, hin=win, wp=wp, hp=wp,
                         ho=ho, wo=wo))
        cin = _CH
    return plan


def _fused_kernel(x_ref, t0, t1, t2, t3, b0, b1, b2, b3,
                  lw, lb, ow, ob, o_ref, p1, p2, p3, *, plan, batch):
    B = batch
    t_refs = (t0, t1, t2, t3)
    b_refs = (b0, b1, b2, b3)
    planes = (None, p1, p2, p3)

    for li in range(1, 4):
        p = planes[li]
        hp = p.shape[0]
        zrow = jnp.zeros((1,) + p.shape[1:], p.dtype)
        p[0:1] = zrow
        p[hp - 1:hp] = zrow

    src = x_ref
    act = None
    for li, g in enumerate(plan):
        k, ho, nc = g["k"], g["ho"], g["wo"] * _CH
        kdim = g["win"] * g["cin"]
        acc = jnp.dot(src[0:ho].reshape(ho * B, kdim), t_refs[li][0],
                      preferred_element_type=jnp.float32)
        for i in range(1, k):
            acc = acc + jnp.dot(src[i:i + ho].reshape(ho * B, kdim),
                                t_refs[li][i],
                                preferred_element_type=jnp.float32)
        act = jnp.maximum(acc + b_refs[li][...], 0.0)
        if li + 1 < 4:
            nxt = planes[li + 1]
            nxt[1:1 + ho] = act.astype(nxt.dtype).reshape(ho, B, nc)
            src = nxt

    glast = plan[-1]
    ho4, nc4 = glast["ho"], glast["wo"] * _CH
    a3 = act.astype(jnp.bfloat16).reshape(ho4, B, nc4)
    y = jnp.dot(a3[0], lw[0], preferred_element_type=jnp.float32)
    for h in range(1, ho4):
        y = y + jnp.dot(a3[h], lw[h], preferred_element_type=jnp.float32)
    feat = jax.nn.sigmoid(y + lb[...])

    bh = B // 2
    d = jnp.abs(feat[0:bh] - feat[bh:B])
    o_ref[...] = (jnp.sum(d * ow[...], axis=1, keepdims=True)
                  + ob[...]).astype(o_ref.dtype)


def _forward(x1, x2, t0, t1, t2, t3, b0, b1, b2, b3, lin_w3, lin_b,
             out_w, out_b):
    n = x1.shape[0]
    plan = _geometry([t0.shape, t1.shape, t2.shape, t3.shape])
    g0, glast = plan[0], plan[-1]
    hidden = lin_w3.shape[-1]

    B = 32 if (2 * n) % 32 == 0 else 2 * n
    bh = B // 2
    nb = (2 * n) // B

    x1p = x1[:, 0, :, :].reshape(nb, bh, g0["hin"], g0["win"])
    x2p = x2[:, 0, :, :].reshape(nb, bh, g0["hin"], g0["win"])
    x_all = jnp.concatenate([x1p, x2p], axis=1).reshape(
        2 * n, g0["hin"], g0["win"])
    x_t = jnp.pad(jnp.transpose(x_all, (1, 0, 2)),
                  ((_PAD, _PAD), (0, 0), (0, 0))).astype(jnp.bfloat16)

    tws = []
    for t, g in zip((t0, t1, t2, t3), plan):
        c = g["cin"]
        tws.append(t[:, c * _PAD:c * _PAD + g["win"] * c, :]
                   .astype(jnp.bfloat16))
    lwb = lin_w3.astype(jnp.bfloat16)
    ow_row = out_w.reshape(1, hidden)

    in_specs = [pl.BlockSpec((g0["hp"], B, g0["win"]), lambda i: (0, i, 0))]
    for t in tws:
        in_specs.append(pl.BlockSpec(t.shape, lambda i: (0, 0, 0)))
    for b in (b0, b1, b2, b3):
        in_specs.append(pl.BlockSpec(b.shape, lambda i: (0, 0)))
    in_specs.append(pl.BlockSpec(lwb.shape, lambda i: (0, 0, 0)))
    in_specs.append(pl.BlockSpec(lin_b.shape, lambda i: (0, 0)))
    in_specs.append(pl.BlockSpec(ow_row.shape, lambda i: (0, 0)))
    in_specs.append(pl.BlockSpec(out_b.shape, lambda i: (0, 0)))

    scratch = [pltpu.VMEM((g["hp"], B, g["win"] * g["cin"]), jnp.bfloat16)
               for g in plan[1:]]

    out = pl.pallas_call(
        functools.partial(_fused_kernel, plan=plan, batch=B),
        out_shape=jax.ShapeDtypeStruct((n, 1), jnp.float32),
        grid=(nb,),
        in_specs=in_specs,
        out_specs=pl.BlockSpec((bh, 1), lambda i: (i, 0)),
        scratch_shapes=scratch,
        compiler_params=pltpu.CompilerParams(
            dimension_semantics=("parallel",)),
    )(x_t, *tws, b0, b1, b2, b3, lwb, lin_b, ow_row, out_b)
    return out


def kernel(x1, x2, t0, t1, t2, t3, b0, b1, b2, b3, lin_w3, lin_b,
           out_w, out_b):
    devs = jax.devices()
    if len(devs) < 2 or x1.shape[0] % 2:
        return _forward(x1, x2, t0, t1, t2, t3, b0, b1, b2, b3,
                        lin_w3, lin_b, out_w, out_b)
    mesh = Mesh(np.array(devs[:2]), ("d",))
    rep = (P(),) * 12
    f = jax.shard_map(_forward, mesh=mesh,
                      in_specs=(P("d"), P("d")) + rep,
                      out_specs=P("d"), check_vma=False)
    return f(x1, x2, t0, t1, t2, t3, b0, b1, b2, b3,
             lin_w3, lin_b, out_w, out_b)


# speedup vs baseline: 2.3089x; 2.3089x over previous
import functools

import numpy as np
import jax
import jax.numpy as jnp
from jax.experimental import pallas as pl
from jax.experimental.pallas import tpu as pltpu
from jax.sharding import Mesh, PartitionSpec as P

_CH = 8
_PAD = 1


def _geometry(t_shapes):
    plan = []
    cin = 1
    for (k, wpcin, wocout) in t_shapes:
        wp = wpcin // cin
        wo = wocout // _CH
        win = wp - 2 * _PAD
        ho = wp - k + 1
        assert ho == wo
        plan.append(dict(k=k, cin=cin, win=win, hin=win, wp=wp, hp=wp,
                         ho=ho, wo=wo))
        cin = _CH
    return plan


def _fused_kernel(x_ref, t0, t1, t2, t3, b0, b1, b2, b3,
                  lw, lb, ow, ob, o_ref, p1, p2, p3, *, plan, batch):
    B = batch
    t_refs = (t0, t1, t2, t3)
    b_refs = (b0, b1, b2, b3)
    planes = (None, p1, p2, p3)

    for li in range(1, 4):
        p = planes[li]
        hp = p.shape[0]
        zrow = jnp.zeros((1,) + p.shape[1:], p.dtype)
        p[0:1] = zrow
        p[hp - 1:hp] = zrow

    src = x_ref
    act = None
    for li, g in enumerate(plan):
        k, ho, nc = g["k"], g["ho"], g["wo"] * _CH
        kdim = g["win"] * g["cin"]
        acc = jnp.dot(src[0:ho].reshape(ho * B, kdim), t_refs[li][0],
                      preferred_element_type=jnp.float32)
        for i in range(1, k):
            acc = acc + jnp.dot(src[i:i + ho].reshape(ho * B, kdim),
                                t_refs[li][i],
                                preferred_element_type=jnp.float32)
        act = jnp.maximum(acc + b_refs[li][...], 0.0)
        if li + 1 < 4:
            nxt = planes[li + 1]
            nxt[1:1 + ho] = act.astype(nxt.dtype).reshape(ho, B, nc)
            src = nxt

    glast = plan[-1]
    ho4, nc4 = glast["ho"], glast["wo"] * _CH
    a3 = act.astype(jnp.bfloat16).reshape(ho4, B, nc4)
    y = jnp.dot(a3[0], lw[0], preferred_element_type=jnp.float32)
    for h in range(1, ho4):
        y = y + jnp.dot(a3[h], lw[h], preferred_element_type=jnp.float32)
    feat = jax.nn.sigmoid(y + lb[...])

    bh = B // 2
    d = jnp.abs(feat[0:bh] - feat[bh:B])
    o_ref[...] = (jnp.sum(d * ow[...], axis=1, keepdims=True)
                  + ob[...]).astype(o_ref.dtype)


def _forward(x1, x2, t0, t1, t2, t3, b0, b1, b2, b3, lin_w3, lin_b,
             out_w, out_b):
    n = x1.shape[0]
    plan = _geometry([t0.shape, t1.shape, t2.shape, t3.shape])
    g0, glast = plan[0], plan[-1]
    hidden = lin_w3.shape[-1]

    B = 64 if (2 * n) % 64 == 0 else 2 * n
    bh = B // 2
    nb = (2 * n) // B

    x1p = x1[:, 0, :, :].reshape(nb, bh, g0["hin"], g0["win"])
    x2p = x2[:, 0, :, :].reshape(nb, bh, g0["hin"], g0["win"])
    x_all = jnp.concatenate([x1p, x2p], axis=1).reshape(
        2 * n, g0["hin"], g0["win"])
    x_t = jnp.pad(jnp.transpose(x_all, (1, 0, 2)),
                  ((_PAD, _PAD), (0, 0), (0, 0))).astype(jnp.bfloat16)

    tws = []
    for t, g in zip((t0, t1, t2, t3), plan):
        c = g["cin"]
        tws.append(t[:, c * _PAD:c * _PAD + g["win"] * c, :]
                   .astype(jnp.bfloat16))
    lwb = lin_w3.astype(jnp.bfloat16)
    ow_row = out_w.reshape(1, hidden)

    in_specs = [pl.BlockSpec((g0["hp"], B, g0["win"]), lambda i: (0, i, 0))]
    for t in tws:
        in_specs.append(pl.BlockSpec(t.shape, lambda i: (0, 0, 0)))
    for b in (b0, b1, b2, b3):
        in_specs.append(pl.BlockSpec(b.shape, lambda i: (0, 0)))
    in_specs.append(pl.BlockSpec(lwb.shape, lambda i: (0, 0, 0)))
    in_specs.append(pl.BlockSpec(lin_b.shape, lambda i: (0, 0)))
    in_specs.append(pl.BlockSpec(ow_row.shape, lambda i: (0, 0)))
    in_specs.append(pl.BlockSpec(out_b.shape, lambda i: (0, 0)))

    scratch = [pltpu.VMEM((g["hp"], B, g["win"] * g["cin"]), jnp.bfloat16)
               for g in plan[1:]]

    out = pl.pallas_call(
        functools.partial(_fused_kernel, plan=plan, batch=B),
        out_shape=jax.ShapeDtypeStruct((n, 1), jnp.float32),
        grid=(nb,),
        in_specs=in_specs,
        out_specs=pl.BlockSpec((bh, 1), lambda i: (i, 0)),
        scratch_shapes=scratch,
        compiler_params=pltpu.CompilerParams(
            dimension_semantics=("parallel",)),
    )(x_t, *tws, b0, b1, b2, b3, lwb, lin_b, ow_row, out_b)
    return out


def kernel(x1, x2, t0, t1, t2, t3, b0, b1, b2, b3, lin_w3, lin_b,
           out_w, out_b):
    return _forward(x1, x2, t0, t1, t2, t3, b0, b1, b2, b3,
                    lin_w3, lin_b, out_w, out_b)


# speedup vs baseline: 2.4671x; 1.0685x over previous
import functools

import jax
import jax.numpy as jnp
from jax.experimental import pallas as pl
from jax.experimental.pallas import tpu as pltpu

_CH = 8
_PAD = 1


def _geometry(t_shapes):
    plan = []
    cin = 1
    for (k, wpcin, wocout) in t_shapes:
        wp = wpcin // cin
        wo = wocout // _CH
        win = wp - 2 * _PAD
        ho = wp - k + 1
        assert ho == wo
        plan.append(dict(k=k, cin=cin, win=win, hin=win, wp=wp, hp=wp,
                         ho=ho, wo=wo))
        cin = _CH
    return plan


def _slab_width(g, li):
    K = g["wp"] if li == 0 else g["win"] * g["cin"]
    return K, -(-K // 128) * 128


def _fused_kernel(x_ref, w0, w1, w2, w3, b0, b1, b2, b3,
                  lw, lb, ow, ob, o_ref, il0, il1, il2, il3, *,
                  plan, batch):
    B = batch
    ws = (w0, w1, w2, w3)
    bs = (b0, b1, b2, b3)
    ils = (il0, il1, il2, il3)

    act = x_ref[...]
    prev_rows = plan[0]["hp"]
    for li, g in enumerate(plan):
        k, ho = g["k"], g["ho"]
        K, kp = _slab_width(g, li)
        il = ils[li]
        zgap = (jnp.zeros((ho, B, kp - K), il.dtype) if kp > K else None)
        for i in range(k):
            col = i * kp
            if li == 0:
                il[0:ho, :, col:col + K] = act[i:i + ho]
                if zgap is not None:
                    il[0:ho, :, col + K:col + kp] = zgap
            else:
                lo = max(0, 1 - i)
                hi = min(ho, prev_rows + 1 - i)
                il[lo:hi, :, col:col + K] = act[i + lo - 1:i + hi - 1]
                if zgap is not None:
                    il[lo:hi, :, col + K:col + kp] = zgap[0:hi - lo]
                if lo > 0:
                    il[0:lo, :, col:col + kp] = jnp.zeros(
                        (lo, B, kp), il.dtype)
                if hi < ho:
                    il[hi:ho, :, col:col + kp] = jnp.zeros(
                        (ho - hi, B, kp), il.dtype)
        nc = g["wo"] * _CH
        acc = jnp.dot(il[0:ho].reshape(ho * B, k * kp), ws[li][...],
                      preferred_element_type=jnp.float32)
        a = jnp.maximum(acc + bs[li][...], 0.0)
        act = a.astype(jnp.bfloat16).reshape(ho, B, nc)
        prev_rows = ho

    ho4 = plan[-1]["ho"]
    y = jnp.dot(act[0], lw[0], preferred_element_type=jnp.float32)
    for h in range(1, ho4):
        y = y + jnp.dot(act[h], lw[h], preferred_element_type=jnp.float32)
    feat = jax.nn.sigmoid(y + lb[...])

    bh = B // 2
    d = jnp.abs(feat[0:bh] - feat[bh:B])
    o_ref[...] = (jnp.sum(d * ow[...], axis=1, keepdims=True)
                  + ob[...]).astype(o_ref.dtype)


def kernel(x1, x2, t0, t1, t2, t3, b0, b1, b2, b3, lin_w3, lin_b,
           out_w, out_b):
    n = x1.shape[0]
    plan = _geometry([t0.shape, t1.shape, t2.shape, t3.shape])
    g0 = plan[0]
    hidden = lin_w3.shape[-1]

    B = 32 if (2 * n) % 32 == 0 else 2 * n
    bh = B // 2
    nb = (2 * n) // B

    x1p = x1[:, 0, :, :].reshape(nb, bh, g0["hin"], g0["win"])
    x2p = x2[:, 0, :, :].reshape(nb, bh, g0["hin"], g0["win"])
    x_all = jnp.concatenate([x1p, x2p], axis=1).reshape(
        2 * n, g0["hin"], g0["win"])
    x_t = jnp.pad(jnp.transpose(x_all, (1, 0, 2)),
                  ((_PAD, _PAD), (0, 0), (_PAD, _PAD))).astype(jnp.bfloat16)

    tws = []
    for li, (t, g) in enumerate(zip((t0, t1, t2, t3), plan)):
        K, kp = _slab_width(g, li)
        if li == 0:
            w = t
        else:
            c = g["cin"]
            w = t[:, c * _PAD:c * _PAD + K, :]
        w = jnp.pad(w, ((0, 0), (0, kp - K), (0, 0)))
        tws.append(w.reshape(g["k"] * kp, t.shape[2]).astype(jnp.bfloat16))
    lwb = lin_w3.astype(jnp.bfloat16)
    ow_row = out_w.reshape(1, hidden)

    in_specs = [pl.BlockSpec((g0["hp"], B, g0["wp"]), lambda i: (0, i, 0))]
    for w in tws:
        in_specs.append(pl.BlockSpec(w.shape, lambda i: (0, 0)))
    for b in (b0, b1, b2, b3):
        in_specs.append(pl.BlockSpec(b.shape, lambda i: (0, 0)))
    in_specs.append(pl.BlockSpec(lwb.shape, lambda i: (0, 0, 0)))
    in_specs.append(pl.BlockSpec(lin_b.shape, lambda i: (0, 0)))
    in_specs.append(pl.BlockSpec(ow_row.shape, lambda i: (0, 0)))
    in_specs.append(pl.BlockSpec(out_b.shape, lambda i: (0, 0)))

    scratch = []
    for li, g in enumerate(plan):
        K, kp = _slab_width(g, li)
        scratch.append(pltpu.VMEM((g["ho"], B, g["k"] * kp), jnp.bfloat16))

    out = pl.pallas_call(
        functools.partial(_fused_kernel, plan=plan, batch=B),
        out_shape=jax.ShapeDtypeStruct((n, 1), jnp.float32),
        grid=(nb,),
        in_specs=in_specs,
        out_specs=pl.BlockSpec((bh, 1), lambda i: (i, 0)),
        scratch_shapes=scratch,
        compiler_params=pltpu.CompilerParams(
            dimension_semantics=("parallel",)),
    )(x_t, *tws, b0, b1, b2, b3, lwb, lin_b, ow_row, out_b)
    return out


# speedup vs baseline: 2.7026x; 1.0954x over previous
import functools

import jax
import jax.numpy as jnp
from jax.experimental import pallas as pl
from jax.experimental.pallas import tpu as pltpu

_CH = 8
_PAD = 1


def _geometry(t_shapes):
    plan = []
    cin = 1
    for (k, wpcin, wocout) in t_shapes:
        wp = wpcin // cin
        wo = wocout // _CH
        win = wp - 2 * _PAD
        ho = wp - k + 1
        assert ho == wo
        plan.append(dict(k=k, cin=cin, win=win, hin=win, wp=wp, hp=wp,
                         ho=ho, wo=wo))
        cin = _CH
    return plan


def _slab_width(g, li):
    K = g["win"] * g["cin"]
    if li == 0:
        return K, K
    return K, -(-K // 128) * 128


def _fused_kernel(x1_ref, x2_ref, w0, w1, w2, w3, b0, b1, b2, b3,
                  lw, lb, ow, ob, o_ref, il0, il1, il2, il3, *,
                  plan, batch):
    B = batch
    bh = B // 2
    ws = (w0, w1, w2, w3)
    bs = (b0, b1, b2, b3)
    ils = (il0, il1, il2, il3)

    act = None
    prev_rows = plan[0]["hin"]
    for li, g in enumerate(plan):
        k, ho = g["k"], g["ho"]
        K, kp = _slab_width(g, li)
        il = ils[li]
        zgap = (jnp.zeros((ho, B, kp - K), il.dtype) if kp > K else None)
        for i in range(k):
            col = i * kp
            lo = max(0, 1 - i)
            hi = min(ho, prev_rows + 1 - i)
            if li == 0:
                il[lo:hi, 0:bh, col:col + K] = x1_ref[i + lo - 1:i + hi - 1]
                il[lo:hi, bh:B, col:col + K] = x2_ref[i + lo - 1:i + hi - 1]
            else:
                il[lo:hi, :, col:col + K] = act[i + lo - 1:i + hi - 1]
            if zgap is not None:
                il[lo:hi, :, col + K:col + kp] = zgap[0:hi - lo]
            if lo > 0:
                il[0:lo, :, col:col + kp] = jnp.zeros((lo, B, kp), il.dtype)
            if hi < ho:
                il[hi:ho, :, col:col + kp] = jnp.zeros(
                    (ho - hi, B, kp), il.dtype)
        nc = g["wo"] * _CH
        acc = jnp.dot(il[0:ho].reshape(ho * B, k * kp), ws[li][...],
                      preferred_element_type=jnp.float32)
        a = jnp.maximum(acc + bs[li][...], 0.0)
        act = a.astype(jnp.bfloat16).reshape(ho, B, nc)
        prev_rows = ho

    ho4 = plan[-1]["ho"]
    y = jnp.dot(act[0], lw[0], preferred_element_type=jnp.float32)
    for h in range(1, ho4):
        y = y + jnp.dot(act[h], lw[h], preferred_element_type=jnp.float32)
    feat = jax.nn.sigmoid(y + lb[...])

    bh = B // 2
    d = jnp.abs(feat[0:bh] - feat[bh:B])
    o_ref[...] = (jnp.sum(d * ow[...], axis=1, keepdims=True)
                  + ob[...]).astype(o_ref.dtype)


def kernel(x1, x2, t0, t1, t2, t3, b0, b1, b2, b3, lin_w3, lin_b,
           out_w, out_b):
    n = x1.shape[0]
    plan = _geometry([t0.shape, t1.shape, t2.shape, t3.shape])
    g0 = plan[0]
    hidden = lin_w3.shape[-1]

    B = 32 if (2 * n) % 32 == 0 else 2 * n
    bh = B // 2
    nb = (2 * n) // B

    x1t = jnp.transpose(x1[:, 0, :, :].astype(jnp.bfloat16), (1, 0, 2))
    x2t = jnp.transpose(x2[:, 0, :, :].astype(jnp.bfloat16), (1, 0, 2))

    tws = []
    for li, (t, g) in enumerate(zip((t0, t1, t2, t3), plan)):
        K, kp = _slab_width(g, li)
        c = g["cin"]
        w = t[:, c * _PAD:c * _PAD + K, :]
        if kp > K:
            w = jnp.pad(w, ((0, 0), (0, kp - K), (0, 0)))
        tws.append(w.reshape(g["k"] * kp, t.shape[2]).astype(jnp.bfloat16))
    lwb = lin_w3.astype(jnp.bfloat16)
    ow_row = out_w.reshape(1, hidden)

    in_specs = [
        pl.BlockSpec((g0["hin"], bh, g0["win"]), lambda i: (0, i, 0)),
        pl.BlockSpec((g0["hin"], bh, g0["win"]), lambda i: (0, i, 0)),
    ]
    for w in tws:
        in_specs.append(pl.BlockSpec(w.shape, lambda i: (0, 0)))
    for b in (b0, b1, b2, b3):
        in_specs.append(pl.BlockSpec(b.shape, lambda i: (0, 0)))
    in_specs.append(pl.BlockSpec(lwb.shape, lambda i: (0, 0, 0)))
    in_specs.append(pl.BlockSpec(lin_b.shape, lambda i: (0, 0)))
    in_specs.append(pl.BlockSpec(ow_row.shape, lambda i: (0, 0)))
    in_specs.append(pl.BlockSpec(out_b.shape, lambda i: (0, 0)))

    scratch = []
    for li, g in enumerate(plan):
        K, kp = _slab_width(g, li)
        scratch.append(pltpu.VMEM((g["ho"], B, g["k"] * kp), jnp.bfloat16))

    out = pl.pallas_call(
        functools.partial(_fused_kernel, plan=plan, batch=B),
        out_shape=jax.ShapeDtypeStruct((n, 1), jnp.float32),
        grid=(nb,),
        in_specs=in_specs,
        out_specs=pl.BlockSpec((bh, 1), lambda i: (i, 0)),
        scratch_shapes=scratch,
        compiler_params=pltpu.CompilerParams(
            dimension_semantics=("parallel",)),
    )(x1t, x2t, *tws, b0, b1, b2, b3, lwb, lin_b, ow_row, out_b)
    return out
